# baseline (speedup 1.0000x reference)
import functools

import jax
import jax.numpy as jnp
from jax import lax
from jax.experimental import pallas as pl
from jax.experimental.pallas import tpu as pltpu
from jax.experimental.pallas import tpu_sc as plsc

NH = 512
DT = 1.0
N_NODES = 100000
N_ROWS = NH + 2

BLOCK_W = 6144

LANES = 128
N_TILES = N_NODES // LANES
SC_COLS = N_TILES * LANES
TAIL = N_NODES - SC_COLS
TAIL_OFF = SC_COLS - 16 * BLOCK_W
BIG = 25 * LANES
SMALL = 24 * LANES
N_BIG = 13


def _tc_kernel(t_ref, buf_ref, w_ref, out_ref, tail_ref):
    j = pl.program_id(0)
    tt = t_ref[0, 0]
    out_ref[...] = buf_ref[...]
    x = buf_ref[pl.ds(NH + tt, 1), :]
    r0 = buf_ref[pl.ds(tt, 1), :]
    r1 = buf_ref[pl.ds(tt + 1, 1), :]
    w = w_ref[...]
    d1 = 0.1 * (r0 - x)
    xi = jnp.tanh(x + DT * d1 + w)
    d2 = 0.1 * (r1 - xi)
    nx = jnp.tanh(x + DT * 0.5 * (d1 + d2) + w)
    out_ref[pl.ds(NH + tt + 1, 1), :] = nx

    @pl.when(j == pl.num_programs(0) - 1)
    def _tail():
        tail_ref[...] = nx[:, TAIL_OFF:TAIL_OFF + TAIL]


def _tanh16(z):
    return 1.0 - 2.0 / (jnp.exp(2.0 * z) + 1.0)


def _sc_nx(t, buf, w2d):
    vector_mesh = plsc.VectorSubcoreMesh(core_axis_name="c",
                                         subcore_axis_name="s")

    @functools.partial(
        pl.kernel,
        out_type=jax.ShapeDtypeStruct((1, SC_COLS), jnp.float32),
        mesh=vector_mesh,
        scratch_types=[
            pltpu.VMEM((1, 128), jnp.int32),
            pltpu.VMEM((1, BIG), jnp.float32),
            pltpu.VMEM((1, BIG), jnp.float32),
            pltpu.VMEM((1, BIG), jnp.float32),
            pltpu.VMEM((1, BIG), jnp.float32),
            pltpu.VMEM((1, BIG), jnp.float32),
        ],
    )
    def nx_kernel(t_hbm, buf_hbm, w_hbm, nx_hbm,
                  t_s, x_s, r0_s, r1_s, w_s, nx_s):
        u = lax.axis_index("c") * 16 + lax.axis_index("s")
        col = jnp.where(u < N_BIG, BIG * u,
                        N_BIG * BIG + SMALL * (u - N_BIG))
        pltpu.sync_copy(t_hbm, t_s)
        tt = t_s.at[0, pl.ds(0, 16)][...][0]

        def work(sz):
            cols = pl.ds(col, sz)
            dst = pl.ds(0, sz)
            pltpu.sync_copy(buf_hbm.at[pl.ds(NH + tt, 1), cols],
                            x_s.at[:, dst])
            pltpu.sync_copy(buf_hbm.at[pl.ds(tt, 1), cols],
                            r0_s.at[:, dst])
            pltpu.sync_copy(buf_hbm.at[pl.ds(tt + 1, 1), cols],
                            r1_s.at[:, dst])
            pltpu.sync_copy(w_hbm.at[:, cols], w_s.at[:, dst])

            @pl.loop(0, sz, step=16)
            def _chunk(i):
                sl = (slice(None), pl.ds(i, 16))
                x = x_s.at[sl][...]
                w = w_s.at[sl][...]
                d1 = 0.1 * (r0_s.at[sl][...] - x)
                xi = _tanh16(x + DT * d1 + w)
                d2 = 0.1 * (r1_s.at[sl][...] - xi)
                nx_s.at[sl][...] = _tanh16(x + DT * 0.5 * (d1 + d2) + w)

            pltpu.sync_copy(nx_s.at[:, dst], nx_hbm.at[:, cols])

        @pl.when(u < N_BIG)
        def _big():
            work(BIG)

        @pl.when(u >= N_BIG)
        def _small():
            work(SMALL)

    t128 = jnp.broadcast_to(t, (1, 128))
    return nx_kernel(t128, buf, w2d)


@functools.partial(jax.jit, static_argnames=())
def kernel(buf, dWt, t):
    w2d = dWt.reshape(1, N_NODES)
    nx_sc = _sc_nx(t, buf, w2d)
    grid = (pl.cdiv(N_NODES, BLOCK_W),)
    out_buf, nx_tail = pl.pallas_call(
        _tc_kernel,
        grid=grid,
        in_specs=[
            pl.BlockSpec(memory_space=pltpu.SMEM),
            pl.BlockSpec((N_ROWS, BLOCK_W), lambda j: (0, j)),
            pl.BlockSpec((1, BLOCK_W), lambda j: (0, j)),
        ],
        out_specs=[
            pl.BlockSpec((N_ROWS, BLOCK_W), lambda j: (0, j)),
            pl.BlockSpec((1, TAIL), lambda j: (0, 0)),
        ],
        out_shape=[
            jax.ShapeDtypeStruct((N_ROWS, N_NODES), jnp.float32),
            jax.ShapeDtypeStruct((1, TAIL), jnp.float32),
        ],
    )(t, buf, w2d)
    nx = jnp.concatenate([nx_sc, nx_tail], axis=1)
    return (out_buf, nx.reshape(N_NODES))


# speedup vs baseline: 1.0006x; 1.0006x over previous
import functools

import jax
import jax.numpy as jnp
from jax import lax
from jax.experimental import pallas as pl
from jax.experimental.pallas import tpu as pltpu
from jax.experimental.pallas import tpu_sc as plsc

NH = 512
DT = 1.0
N_NODES = 100000
N_ROWS = NH + 2

BLOCK_W = 6144

LANES = 128
N_TILES = N_NODES // LANES
SC_COLS = N_TILES * LANES
TAIL = N_NODES - SC_COLS
TAIL_OFF = SC_COLS - 16 * BLOCK_W
BIG = 25 * LANES
SMALL = 24 * LANES
N_BIG = 13


def _tc_kernel(t_ref, buf_ref, w_ref, out_ref, tail_ref):
    j = pl.program_id(0)
    tt = t_ref[0, 0]
    out_ref[...] = buf_ref[...]
    x = buf_ref[pl.ds(NH + tt, 1), :]
    r0 = buf_ref[pl.ds(tt, 1), :]
    r1 = buf_ref[pl.ds(tt + 1, 1), :]
    w = w_ref[...]
    d1 = 0.1 * (r0 - x)
    xi = jnp.tanh(x + DT * d1 + w)
    d2 = 0.1 * (r1 - xi)
    nx = jnp.tanh(x + DT * 0.5 * (d1 + d2) + w)
    out_ref[pl.ds(NH + tt + 1, 1), :] = nx

    @pl.when(j == pl.num_programs(0) - 1)
    def _tail():
        tail_ref[...] = nx[:, TAIL_OFF:TAIL_OFF + TAIL]


def _tanh16(z):
    return 1.0 - 2.0 / (jnp.exp(2.0 * z) + 1.0)


def _sc_nx(t, buf, w2d):
    vector_mesh = plsc.VectorSubcoreMesh(core_axis_name="c",
                                         subcore_axis_name="s")

    @functools.partial(
        pl.kernel,
        out_type=jax.ShapeDtypeStruct((1, SC_COLS), jnp.float32),
        mesh=vector_mesh,
        scratch_types=[
            pltpu.VMEM((1, 128), jnp.int32),
            pltpu.VMEM((1, BIG), jnp.float32),
            pltpu.VMEM((1, BIG), jnp.float32),
            pltpu.VMEM((1, BIG), jnp.float32),
            pltpu.VMEM((1, BIG), jnp.float32),
            pltpu.VMEM((1, BIG), jnp.float32),
        ],
    )
    def nx_kernel(t_hbm, buf_hbm, w_hbm, nx_hbm,
                  t_s, x_s, r0_s, r1_s, w_s, nx_s):
        u = lax.axis_index("c") * 16 + lax.axis_index("s")
        col = jnp.where(u < N_BIG, BIG * u,
                        N_BIG * BIG + SMALL * (u - N_BIG))
        pltpu.sync_copy(t_hbm, t_s)
        tt = t_s.at[0, pl.ds(0, 16)][...][0]

        def work(sz):
            cols = pl.ds(col, sz)
            dst = pl.ds(0, sz)
            pltpu.sync_copy(buf_hbm.at[pl.ds(NH + tt, 1), cols],
                            x_s.at[:, dst])
            pltpu.sync_copy(buf_hbm.at[pl.ds(tt, 1), cols],
                            r0_s.at[:, dst])
            pltpu.sync_copy(buf_hbm.at[pl.ds(tt + 1, 1), cols],
                            r1_s.at[:, dst])
            pltpu.sync_copy(w_hbm.at[:, cols], w_s.at[:, dst])

            @pl.loop(0, sz, step=16)
            def _chunk(i):
                sl = (slice(None), pl.ds(i, 16))
                x = x_s.at[sl][...]
                w = w_s.at[sl][...]
                d1 = 0.1 * (r0_s.at[sl][...] - x)
                xi = _tanh16(x + DT * d1 + w)
                d2 = 0.1 * (r1_s.at[sl][...] - xi)
                nx_s.at[sl][...] = _tanh16(x + DT * 0.5 * (d1 + d2) + w)

            pltpu.sync_copy(nx_s.at[:, dst], nx_hbm.at[:, cols])

        @pl.when(u < N_BIG)
        def _big():
            work(BIG)

        @pl.when(u >= N_BIG)
        def _small():
            work(SMALL)

    t128 = jnp.broadcast_to(t, (1, 128))
    return nx_kernel(t128, buf, w2d)


@functools.partial(jax.jit, static_argnames=())
def kernel(buf, dWt, t):
    w2d = dWt.reshape(1, N_NODES)
    grid = (pl.cdiv(N_NODES, BLOCK_W),)
    out_buf, nx_tail = pl.pallas_call(
        _tc_kernel,
        grid=grid,
        in_specs=[
            pl.BlockSpec(memory_space=pltpu.SMEM),
            pl.BlockSpec((N_ROWS, BLOCK_W), lambda j: (0, j)),
            pl.BlockSpec((1, BLOCK_W), lambda j: (0, j)),
        ],
        out_specs=[
            pl.BlockSpec((N_ROWS, BLOCK_W), lambda j: (0, j)),
            pl.BlockSpec((1, TAIL), lambda j: (0, 0)),
        ],
        out_shape=[
            jax.ShapeDtypeStruct((N_ROWS, N_NODES), jnp.float32),
            jax.ShapeDtypeStruct((1, TAIL), jnp.float32),
        ],
    )(t, buf, w2d)
    nx_sc = _sc_nx(t, buf, w2d)
    nx = jnp.concatenate([nx_sc, nx_tail], axis=1)
    return (out_buf, nx.reshape(N_NODES))


# speedup vs baseline: 1.1235x; 1.1229x over previous
import functools

import jax
import jax.numpy as jnp
from jax.experimental import pallas as pl
from jax.experimental.pallas import tpu as pltpu

NH = 512
DT = 1.0
N_NODES = 100000
N_ROWS = NH + 2

BLOCK_W = 6144


def _step_kernel(t_ref, buf_ref, w_ref, out_ref, nx_ref):
    tt = t_ref[0, 0]
    out_ref[...] = buf_ref[...]
    x = buf_ref[pl.ds(NH + tt, 1), :]
    r0 = buf_ref[pl.ds(tt, 1), :]
    r1 = buf_ref[pl.ds(tt + 1, 1), :]
    w = w_ref[...]
    d1 = 0.1 * (r0 - x)
    xi = jnp.tanh(x + DT * d1 + w)
    d2 = 0.1 * (r1 - xi)
    nx = jnp.tanh(x + DT * 0.5 * (d1 + d2) + w)
    out_ref[pl.ds(NH + tt + 1, 1), :] = nx
    nx_ref[...] = nx


@functools.partial(jax.jit, static_argnames=())
def kernel(buf, dWt, t):
    w2d = dWt.reshape(1, N_NODES)
    grid = (pl.cdiv(N_NODES, BLOCK_W),)
    out_buf, nx2d = pl.pallas_call(
        _step_kernel,
        grid=grid,
        in_specs=[
            pl.BlockSpec(memory_space=pltpu.SMEM),
            pl.BlockSpec((N_ROWS, BLOCK_W), lambda j: (0, j)),
            pl.BlockSpec((1, BLOCK_W), lambda j: (0, j)),
        ],
        out_specs=[
            pl.BlockSpec((N_ROWS, BLOCK_W), lambda j: (0, j)),
            pl.BlockSpec((1, BLOCK_W), lambda j: (0, j)),
        ],
        out_shape=[
            jax.ShapeDtypeStruct((N_ROWS, N_NODES), jnp.float32),
            jax.ShapeDtypeStruct((1, N_NODES), jnp.float32),
        ],
    )(t, buf, w2d)
    return (out_buf, nx2d.reshape(N_NODES))
